# baseline (speedup 1.0000x reference)
import functools

import jax
import jax.numpy as jnp
from jax import lax
from jax.experimental import pallas as pl
from jax.experimental.pallas import tpu as pltpu
from jax.experimental.pallas import tpu_sc as plsc

_NE = 64
_EPS = 1e-6
_TILE = 4096

_NW = 32
_TPW = 1024
_ZCHUNK = 4096


def _pass1_body(x_ref, w_ref, b_ref, v_ref, e_ref, r_ref, dacc_ref, *,
                num_tiles, capacity):
    i = pl.program_id(0)
    lt = jax.lax.dot_general(
        w_ref[...], x_ref[...], (((1,), (1,)), ((), ())),
        preferred_element_type=jnp.float32)
    lt = lt + b_ref[...]
    m = jnp.max(lt, axis=0, keepdims=True)
    s = jnp.sum(jnp.exp(lt - m), axis=0, keepdims=True)
    v = 1.0 / s
    iota = jax.lax.broadcasted_iota(jnp.int32, (_NE, _TILE), 0)
    e = jnp.min(jnp.where(lt == m, iota, _NE), axis=0, keepdims=True)
    v_ref[0, 0, :] = v[0]
    e_ref[0, 0, :] = e[0]
    contrib = jnp.where(iota == e, v, 0.0)

    @pl.when(i == 0)
    def _():
        dacc_ref[...] = jnp.zeros_like(dacc_ref)

    dacc_ref[...] += contrib

    @pl.when(i == num_tiles - 1)
    def _():
        denom = jnp.sum(dacc_ref[...], axis=1) + _EPS
        r_ref[0, :] = capacity / denom


def _sc_scatter_body(v_hbm, e_hbm, r_hbm, out_hbm,
                     v_v, e_v, rv_v, idx_v, val_v, zbuf, sem_z, sem_s):
    wid = lax.axis_index("s") * 2 + lax.axis_index("c")
    base = wid * _TPW
    pltpu.sync_copy(v_hbm.at[pl.ds(base, _TPW)], v_v)
    pltpu.sync_copy(e_hbm.at[pl.ds(base, _TPW)], e_v)
    pltpu.async_copy(r_hbm.at[e_v], rv_v, sem_s).wait()

    def _zero(i, carry):
        zbuf[pl.ds(i * 16, 16)] = jnp.zeros((16,), jnp.float32)
        return carry

    lax.fori_loop(0, _ZCHUNK // 16, _zero, 0, unroll=4)

    zcopies = [
        pltpu.make_async_copy(
            zbuf, out_hbm.at[pl.ds(base * _NE + k * _ZCHUNK, _ZCHUNK)], sem_z)
        for k in range(_TPW * _NE // _ZCHUNK)
    ]
    for c in zcopies:
        c.start()

    lanes = lax.iota(jnp.int32, 16) * _NE

    def _build(i, carry):
        sl = pl.ds(i * 16, 16)
        idx_v[sl] = (base + i * 16) * _NE + lanes + e_v[sl]
        val_v[sl] = v_v[sl] * rv_v[sl]
        return carry

    lax.fori_loop(0, _TPW // 16, _build, 0)

    for c in zcopies:
        c.wait()
    pltpu.async_copy(val_v, out_hbm.at[idx_v], sem_s).wait()


def kernel(x, w_gate, b_gate):
    n, dim = x.shape
    ne = w_gate.shape[0]
    capacity = float(n)
    num_tiles = n // _TILE
    b2 = b_gate.reshape(ne, 1)

    v3, e3, recip = pl.pallas_call(
        functools.partial(_pass1_body, num_tiles=num_tiles,
                          capacity=capacity),
        grid=(num_tiles,),
        in_specs=[
            pl.BlockSpec((_TILE, dim), lambda i: (i, 0)),
            pl.BlockSpec((ne, dim), lambda i: (0, 0)),
            pl.BlockSpec((ne, 1), lambda i: (0, 0)),
        ],
        out_specs=[
            pl.BlockSpec((1, 1, _TILE), lambda i: (i, 0, 0)),
            pl.BlockSpec((1, 1, _TILE), lambda i: (i, 0, 0)),
            pl.BlockSpec((1, ne), lambda i: (0, 0)),
        ],
        out_shape=[
            jax.ShapeDtypeStruct((num_tiles, 1, _TILE), jnp.float32),
            jax.ShapeDtypeStruct((num_tiles, 1, _TILE), jnp.int32),
            jax.ShapeDtypeStruct((1, ne), jnp.float32),
        ],
        scratch_shapes=[pltpu.VMEM((_NE, _TILE), jnp.float32)],
        compiler_params=pltpu.CompilerParams(
            dimension_semantics=("arbitrary",)),
    )(x, w_gate, b2)

    sc_kernel = functools.partial(
        pl.kernel,
        out_type=jax.ShapeDtypeStruct((n * ne,), jnp.float32),
        mesh=plsc.VectorSubcoreMesh(core_axis_name="c", subcore_axis_name="s"),
        scratch_types=[
            pltpu.VMEM((_TPW,), jnp.float32),
            pltpu.VMEM((_TPW,), jnp.int32),
            pltpu.VMEM((_TPW,), jnp.float32),
            pltpu.VMEM((_TPW,), jnp.int32),
            pltpu.VMEM((_TPW,), jnp.float32),
            pltpu.VMEM((_ZCHUNK,), jnp.float32),
            pltpu.SemaphoreType.DMA,
            pltpu.SemaphoreType.DMA,
        ],
    )(_sc_scatter_body)
    out_flat = sc_kernel(v3.reshape(n), e3.reshape(n), recip.reshape(ne))
    return out_flat.reshape(n, ne)


# speedup vs baseline: 4.5650x; 4.5650x over previous
import functools

import jax
import jax.numpy as jnp
from jax.experimental import pallas as pl
from jax.experimental.pallas import tpu as pltpu

_NE = 64
_EPS = 1e-6
_TILE = 4096


def _pass1_body(x_ref, w_ref, b_ref, v_ref, e_ref, dacc_ref):
    i = pl.program_id(0)
    lt = jax.lax.dot_general(
        w_ref[...], x_ref[...], (((1,), (1,)), ((), ())),
        preferred_element_type=jnp.float32)
    lt = lt + b_ref[...]
    m = jnp.max(lt, axis=0, keepdims=True)
    s = jnp.sum(jnp.exp(lt - m), axis=0, keepdims=True)
    v = 1.0 / s
    iota = jax.lax.broadcasted_iota(jnp.int32, (_NE, _TILE), 0)
    e = jnp.min(jnp.where(lt == m, iota, _NE), axis=0, keepdims=True)
    v_ref[0, 0, :] = v[0]
    e_ref[0, 0, :] = e[0]
    contrib = jnp.where(iota == e, v, 0.0)

    @pl.when(i == 0)
    def _():
        dacc_ref[...] = jnp.zeros_like(dacc_ref)

    dacc_ref[...] += contrib


def _pass2_body(v_ref, e_ref, dacc_ref, o_ref, *, capacity):
    denom = jnp.sum(dacc_ref[...], axis=1, keepdims=True) + _EPS
    recip = capacity / denom
    v = v_ref[0, 0, :][None, :]
    e = e_ref[0, 0, :][None, :]
    iota = jax.lax.broadcasted_iota(jnp.int32, (_NE, _TILE), 0)
    out_t = jnp.where(iota == e, v * recip, 0.0)
    o_ref[...] = out_t.T


def kernel(x, w_gate, b_gate):
    n, dim = x.shape
    ne = w_gate.shape[0]
    capacity = float(n)
    num_tiles = n // _TILE
    b2 = b_gate.reshape(ne, 1)

    v3, e3, dacc = pl.pallas_call(
        _pass1_body,
        grid=(num_tiles,),
        in_specs=[
            pl.BlockSpec((_TILE, dim), lambda i: (i, 0)),
            pl.BlockSpec((ne, dim), lambda i: (0, 0)),
            pl.BlockSpec((ne, 1), lambda i: (0, 0)),
        ],
        out_specs=[
            pl.BlockSpec((1, 1, _TILE), lambda i: (i, 0, 0)),
            pl.BlockSpec((1, 1, _TILE), lambda i: (i, 0, 0)),
            pl.BlockSpec((ne, _TILE), lambda i: (0, 0)),
        ],
        out_shape=[
            jax.ShapeDtypeStruct((num_tiles, 1, _TILE), jnp.float32),
            jax.ShapeDtypeStruct((num_tiles, 1, _TILE), jnp.int32),
            jax.ShapeDtypeStruct((ne, _TILE), jnp.float32),
        ],
        compiler_params=pltpu.CompilerParams(
            dimension_semantics=("arbitrary",)),
    )(x, w_gate, b2)

    out = pl.pallas_call(
        functools.partial(_pass2_body, capacity=capacity),
        grid=(num_tiles,),
        in_specs=[
            pl.BlockSpec((1, 1, _TILE), lambda i: (i, 0, 0)),
            pl.BlockSpec((1, 1, _TILE), lambda i: (i, 0, 0)),
            pl.BlockSpec((ne, _TILE), lambda i: (0, 0)),
        ],
        out_specs=pl.BlockSpec((_TILE, ne), lambda i: (i, 0)),
        out_shape=jax.ShapeDtypeStruct((n, ne), jnp.float32),
        compiler_params=pltpu.CompilerParams(
            dimension_semantics=("arbitrary",)),
    )(v3, e3, dacc)
    return out
